# Initial kernel scaffold; baseline (speedup 1.0000x reference)
import jax
import jax.numpy as jnp
from jax.experimental import pallas as pl


def kernel(x, edge_index, boo_values):
    raise NotImplementedError("write your pallas kernel here")


# speedup vs baseline: 2.1445x; 2.1445x over previous
import functools

import jax
import jax.numpy as jnp
from jax import lax
from jax.experimental import pallas as pl
from jax.experimental.pallas import tpu as pltpu
from jax.experimental.pallas import tpu_sc as plsc

EPS = 0.1
N = 100000
E = 1600000
D = 8
B = 128
NCHUNKS = E // B
NC = 2
NS = 16
NW = NC * NS
ROWS_PER_TILE = 6256
N_PAD = ROWS_PER_TILE * NS
GROUPS = B // 16

_mesh = plsc.VectorSubcoreMesh(core_axis_name="c", subcore_axis_name="s")

_f32 = jnp.float32
_i32 = jnp.int32


def _c16(v):
    return jnp.full((16,), v, dtype=_i32)


def _edge_matvec_chunk(boo_v, h_v, msg_v, transpose):
    ar16 = lax.iota(_i32, 16)

    def group_body(g, carry):
        e = ar16 + g * 16
        hk = [plsc.load_gather(h_v, [e, _c16(k)]) for k in range(D)]
        for d in range(D):
            acc = jnp.zeros((16,), _f32)
            for k in range(D):
                if transpose:
                    bv = plsc.load_gather(boo_v, [e, _c16(k), _c16(d)])
                else:
                    bv = plsc.load_gather(boo_v, [e, _c16(d), _c16(k)])
                acc = acc + bv * hk[k]
            plsc.store_scatter(msg_v, [e, _c16(d)], acc)
        return carry

    lax.fori_loop(0, GROUPS, group_body, 0)


def _make_pass(transpose, two_sources):
    n_src = 2 if two_sources else 1

    def body(*refs):
        srcs = refs[:n_src]
        gidx_hbm, sidx_hbm, boo_hbm, zeros_hbm = refs[n_src:n_src + 4]
        out_hbm = refs[n_src + 4]
        gidx_v, sidx_v, h_v, boo_v, msg_v, shared, sem = refs[n_src + 5:]

        cid = lax.axis_index("c")
        sid = lax.axis_index("s")
        w = sid * NC + cid

        rs = sid * ROWS_PER_TILE
        pltpu.sync_copy(zeros_hbm.at[pl.ds(rs, ROWS_PER_TILE)],
                        shared.at[pl.ds(rs, ROWS_PER_TILE)])
        plsc.subcore_barrier()

        nch = NCHUNKS // NW + jnp.where(w < NCHUNKS % NW, 1, 0)

        def chunk_body(j, carry):
            c = w + j * NW
            e0 = c * B
            pltpu.sync_copy(gidx_hbm.at[pl.ds(e0, B)], gidx_v)
            pltpu.sync_copy(sidx_hbm.at[pl.ds(e0, B)], sidx_v)
            pltpu.async_copy(srcs[0].at[gidx_v], h_v, sem).wait()
            if two_sources:
                pltpu.async_copy(srcs[1].at[gidx_v], h_v, sem, add=True).wait()
            pltpu.sync_copy(boo_hbm.at[pl.ds(e0, B)], boo_v)
            _edge_matvec_chunk(boo_v, h_v, msg_v, transpose)
            pltpu.sync_copy(msg_v, shared.at[sidx_v], add=True)
            return carry

        lax.fori_loop(0, nch, chunk_body, 0)

        plsc.subcore_barrier()
        pltpu.sync_copy(shared.at[pl.ds(rs, ROWS_PER_TILE)],
                        out_hbm.at[cid, pl.ds(rs, ROWS_PER_TILE)])

    return pl.kernel(
        body,
        out_type=jax.ShapeDtypeStruct((NC, N_PAD, D), _f32),
        mesh=_mesh,
        scratch_types=[
            pltpu.VMEM((B,), _i32),
            pltpu.VMEM((B,), _i32),
            pltpu.VMEM((B, D), _f32),
            pltpu.VMEM((B, D, D), _f32),
            pltpu.VMEM((B, D), _f32),
            pltpu.VMEM_SHARED((N_PAD, D), _f32),
            pltpu.SemaphoreType.DMA,
        ],
        compiler_params=pltpu.CompilerParams(
            needs_layout_passes=False, use_tc_tiling_on_sc=False),
    )


_pass1 = _make_pass(transpose=True, two_sources=False)
_pass2 = _make_pass(transpose=False, two_sources=True)


def _combine_body(x_ref, q0_ref, q1_ref, o_ref):
    o_ref[...] = EPS * x_ref[...] + q0_ref[...] + q1_ref[...]


_combine = pl.pallas_call(
    _combine_body,
    out_shape=jax.ShapeDtypeStruct((N * D // 128, 128), _f32),
)


def kernel(x, edge_index, boo_values):
    src = edge_index[0].astype(_i32)
    dst = edge_index[1].astype(_i32)
    zeros = jnp.zeros((N_PAD, D), _f32)
    p = _pass1(x, src, dst, boo_values, zeros)
    q = _pass2(p[0], p[1], dst, src, boo_values, zeros)
    flat = lambda a: a[:N].reshape(N * D // 128, 128)
    out = _combine(x.reshape(N * D // 128, 128), flat(q[0]), flat(q[1]))
    return out.reshape(N, D)


# speedup vs baseline: 6.1264x; 2.8568x over previous
import functools

import jax
import jax.numpy as jnp
from jax import lax
from jax.experimental import pallas as pl
from jax.experimental.pallas import tpu as pltpu
from jax.experimental.pallas import tpu_sc as plsc

EPS = 0.1
N = 100000
E = 1600000
D = 8
B = 128
NCHUNKS = E // B
NC = 2
NS = 16
NW = NC * NS
ROWS_PER_TILE = 6256
N_PAD = ROWS_PER_TILE * NS
GROUPS = B // 16

_mesh = plsc.VectorSubcoreMesh(core_axis_name="c", subcore_axis_name="s")

_f32 = jnp.float32
_i32 = jnp.int32


def _c16(v):
    return jnp.full((16,), v, dtype=_i32)


def _edge_matvec_chunk(boo_v, h_v, msg_v, transpose):
    ar16 = lax.iota(_i32, 16)

    def group_body(g, carry):
        e = ar16 + g * 16
        ve = pl.ds(g * 16, 16)
        hk = [plsc.load_gather(h_v, [e, _c16(k)]) for k in range(D)]
        for d in range(D):
            acc = jnp.zeros((16,), _f32)
            for k in range(D):
                bv = boo_v[k, d, ve] if transpose else boo_v[d, k, ve]
                acc = acc + bv * hk[k]
            plsc.store_scatter(msg_v, [e, _c16(d)], acc)
        return carry

    lax.fori_loop(0, GROUPS, group_body, 0)


def _make_pass(transpose, two_sources):
    n_src = 2 if two_sources else 1

    def body(*refs):
        srcs = refs[:n_src]
        gidx_hbm, sidx_hbm, boo_hbm, zeros_hbm = refs[n_src:n_src + 4]
        out_hbm = refs[n_src + 4]
        gidx_v, sidx_v, h_v, boo_v, msg_v, shared, sem = refs[n_src + 5:]

        cid = lax.axis_index("c")
        sid = lax.axis_index("s")
        w = sid * NC + cid

        rs = sid * ROWS_PER_TILE
        pltpu.sync_copy(zeros_hbm.at[pl.ds(rs, ROWS_PER_TILE)],
                        shared.at[pl.ds(rs, ROWS_PER_TILE)])
        plsc.subcore_barrier()

        nch = NCHUNKS // NW + jnp.where(w < NCHUNKS % NW, 1, 0)

        def chunk_body(j, carry):
            c = w + j * NW
            e0 = c * B
            pltpu.sync_copy(gidx_hbm.at[pl.ds(e0, B)], gidx_v)
            pltpu.sync_copy(sidx_hbm.at[pl.ds(e0, B)], sidx_v)
            pltpu.async_copy(srcs[0].at[gidx_v], h_v, sem).wait()
            if two_sources:
                pltpu.async_copy(srcs[1].at[gidx_v], h_v, sem, add=True).wait()
            pltpu.sync_copy(boo_hbm.at[:, c], boo_v)
            _edge_matvec_chunk(boo_v, h_v, msg_v, transpose)
            pltpu.sync_copy(msg_v, shared.at[sidx_v], add=True)
            return carry

        lax.fori_loop(0, nch, chunk_body, 0)

        plsc.subcore_barrier()
        pltpu.sync_copy(shared.at[pl.ds(rs, ROWS_PER_TILE)],
                        out_hbm.at[cid, pl.ds(rs, ROWS_PER_TILE)])

    return pl.kernel(
        body,
        out_type=jax.ShapeDtypeStruct((NC, N_PAD, D), _f32),
        mesh=_mesh,
        scratch_types=[
            pltpu.VMEM((B,), _i32),
            pltpu.VMEM((B,), _i32),
            pltpu.VMEM((B, D), _f32),
            pltpu.VMEM((D, D, B), _f32),
            pltpu.VMEM((B, D), _f32),
            pltpu.VMEM_SHARED((N_PAD, D), _f32),
            pltpu.SemaphoreType.DMA,
        ],
        compiler_params=pltpu.CompilerParams(
            needs_layout_passes=False, use_tc_tiling_on_sc=False),
    )


_pass1 = _make_pass(transpose=True, two_sources=False)
_pass2 = _make_pass(transpose=False, two_sources=True)


def _combine_body(x_ref, q0_ref, q1_ref, o_ref):
    o_ref[...] = EPS * x_ref[...] + q0_ref[...] + q1_ref[...]


_combine = pl.pallas_call(
    _combine_body,
    out_shape=jax.ShapeDtypeStruct((N * D // 128, 128), _f32),
)


def kernel(x, edge_index, boo_values):
    src = edge_index[0].astype(_i32)
    dst = edge_index[1].astype(_i32)
    zeros = jnp.zeros((N_PAD, D), _f32)
    boo_sc = boo_values.transpose(1, 2, 0).reshape(D, D, NCHUNKS, B)
    boo_sc = boo_sc.transpose(0, 2, 1, 3)
    p = _pass1(x, src, dst, boo_sc, zeros)
    q = _pass2(p[0], p[1], dst, src, boo_sc, zeros)
    flat = lambda a: a[:N].reshape(N * D // 128, 128)
    out = _combine(x.reshape(N * D // 128, 128), flat(q[0]), flat(q[1]))
    return out.reshape(N, D)


# speedup vs baseline: 11.0363x; 1.8014x over previous
import functools

import jax
import jax.numpy as jnp
from jax import lax
from jax.experimental import pallas as pl
from jax.experimental.pallas import tpu as pltpu
from jax.experimental.pallas import tpu_sc as plsc

EPS = 0.1
N = 100000
E = 1600000
D = 8
TILE = 128
NTILES = E // TILE
BT = 4
BE = TILE * BT
NCH = NTILES // BT
NC = 2
NS = 16
NW = NC * NS
ROWS_PER_TILE = 6256
N_PAD = ROWS_PER_TILE * NS

_mesh = plsc.VectorSubcoreMesh(core_axis_name="c", subcore_axis_name="s")

_f32 = jnp.float32
_i32 = jnp.int32


def _c16(v):
    return jnp.full((16,), v, dtype=_i32)


def _edge_matvec_chunk(boo_v, h_v, h2_v, msg_v, transpose):
    ar16 = lax.iota(_i32, 16)

    for t in range(BT):
        def group_body(g, carry, t=t):
            e = ar16 + (t * TILE + g * 16)
            ve = pl.ds(g * 16, 16)
            hk = [plsc.load_gather(h_v, [e, _c16(k)]) for k in range(D)]
            if h2_v is not None:
                hk = [hk[k] + plsc.load_gather(h2_v, [e, _c16(k)])
                      for k in range(D)]
            for d in range(D):
                acc = jnp.zeros((16,), _f32)
                for k in range(D):
                    bv = boo_v[k, t, d, ve] if transpose else boo_v[d, t, k, ve]
                    acc = acc + bv * hk[k]
                plsc.store_scatter(msg_v, [e, _c16(d)], acc)
            return carry

        lax.fori_loop(0, TILE // 16, group_body, 0)


def _make_pass(transpose, two_sources):
    n_src = 2 if two_sources else 1

    def body(*refs):
        srcs = refs[:n_src]
        gidx_hbm, sidx_hbm, boo_hbm, zeros_hbm = refs[n_src:n_src + 4]
        out_hbm = refs[n_src + 4]
        (gidx_v, sidx_v, h_v, h2_v, boo_v, msg_v, shared, sem,
         sem_s) = refs[n_src + 5:]

        cid = lax.axis_index("c")
        sid = lax.axis_index("s")
        w = sid * NC + cid

        rs = sid * ROWS_PER_TILE
        pltpu.sync_copy(zeros_hbm.at[pl.ds(rs, ROWS_PER_TILE)],
                        shared.at[pl.ds(rs, ROWS_PER_TILE)])
        plsc.subcore_barrier()

        nch = NCH // NW + jnp.where(w < NCH % NW, 1, 0)

        def chunk_body(j, carry):
            c = w + j * NW
            ct = c * BT
            pltpu.sync_copy(gidx_hbm.at[pl.ds(ct, BT)], gidx_v)
            pltpu.sync_copy(sidx_hbm.at[pl.ds(ct, BT)], sidx_v)
            gathers = []
            for t in range(BT):
                dst = h_v.at[pl.ds(t * TILE, TILE)]
                gathers.append(
                    pltpu.async_copy(srcs[0].at[gidx_v.at[t]], dst, sem))
            if two_sources:
                for t in range(BT):
                    dst = h2_v.at[pl.ds(t * TILE, TILE)]
                    gathers.append(
                        pltpu.async_copy(srcs[1].at[gidx_v.at[t]], dst, sem))
            pltpu.sync_copy(boo_hbm.at[:, pl.ds(ct, BT)], boo_v)
            for g in gathers:
                g.wait()
            _edge_matvec_chunk(boo_v, h_v, h2_v if two_sources else None,
                               msg_v, transpose)
            scatters = []
            for t in range(BT):
                src = msg_v.at[pl.ds(t * TILE, TILE)]
                scatters.append(
                    pltpu.async_copy(src, shared.at[sidx_v.at[t]], sem_s,
                                     add=True))
            for s in scatters:
                s.wait()
            return carry

        lax.fori_loop(0, nch, chunk_body, 0)

        plsc.subcore_barrier()
        pltpu.sync_copy(shared.at[pl.ds(rs, ROWS_PER_TILE)],
                        out_hbm.at[cid, pl.ds(rs, ROWS_PER_TILE)])

    return pl.kernel(
        body,
        out_type=jax.ShapeDtypeStruct((NC, N_PAD, D), _f32),
        mesh=_mesh,
        scratch_types=[
            pltpu.VMEM((BT, TILE), _i32),
            pltpu.VMEM((BT, TILE), _i32),
            pltpu.VMEM((BE, D), _f32),
            pltpu.VMEM((BE, D), _f32),
            pltpu.VMEM((D, BT, D, TILE), _f32),
            pltpu.VMEM((BE, D), _f32),
            pltpu.VMEM_SHARED((N_PAD, D), _f32),
            pltpu.SemaphoreType.DMA,
            pltpu.SemaphoreType.DMA,
        ],
        compiler_params=pltpu.CompilerParams(
            needs_layout_passes=False, use_tc_tiling_on_sc=False),
    )


_pass1 = _make_pass(transpose=True, two_sources=False)
_pass2 = _make_pass(transpose=False, two_sources=True)


def _combine_body(x_ref, q0_ref, q1_ref, o_ref):
    o_ref[...] = EPS * x_ref[...] + q0_ref[...] + q1_ref[...]


_combine = pl.pallas_call(
    _combine_body,
    out_shape=jax.ShapeDtypeStruct((N * D // 128, 128), _f32),
)


def kernel(x, edge_index, boo_values):
    src = edge_index[0].astype(_i32).reshape(NTILES, TILE)
    dst = edge_index[1].astype(_i32).reshape(NTILES, TILE)
    zeros = jnp.zeros((N_PAD, D), _f32)
    boo_sc = boo_values.transpose(1, 2, 0).reshape(D, D, NTILES, TILE)
    boo_sc = boo_sc.transpose(0, 2, 1, 3)
    p = _pass1(x, src, dst, boo_sc, zeros)
    q = _pass2(p[0], p[1], dst, src, boo_sc, zeros)
    flat = lambda a: a[:N].reshape(N * D // 128, 128)
    out = _combine(x.reshape(N * D // 128, 128), flat(q[0]), flat(q[1]))
    return out.reshape(N, D)
